# Initial kernel scaffold; baseline (speedup 1.0000x reference)
import jax
import jax.numpy as jnp
from jax.experimental import pallas as pl


def kernel(x, edge_index, batch, W1, b1, W2, b2, W3, b3, W4, b4):
    raise NotImplementedError("write your pallas kernel here")


# speedup vs baseline: 29.7906x; 29.7906x over previous
import functools

import jax
import jax.numpy as jnp
from jax import lax
from jax.experimental import pallas as pl
from jax.experimental.pallas import tpu as pltpu
from jax.experimental.pallas import tpu_sc as plsc

NC, NS = 2, 16
NW = NC * NS
CH = 80
KDEPTH = 5
NP = 10240

_mesh = functools.partial(
    plsc.VectorSubcoreMesh, core_axis_name="c", subcore_axis_name="s"
)


def _sc_degree(dst1d):
    e = dst1d.shape[0]
    ew = e // NW
    cw = ew // CH
    rw = NP // NS
    zr = rw // 5

    @functools.partial(
        pl.kernel,
        out_type=jax.ShapeDtypeStruct((NC, NP, 16), jnp.float32),
        mesh=_mesh(),
        compiler_params=pltpu.CompilerParams(use_tc_tiling_on_sc=False),
        scratch_types=[
            pltpu.VMEM((KDEPTH, CH), jnp.int32),
            pltpu.VMEM((CH, 16), jnp.float32),
            pltpu.VMEM((zr, 16), jnp.float32),
            pltpu.MemorySpace.VMEM_SHARED((NP, 16), jnp.float32),
            pltpu.SemaphoreType.DMA,
        ],
    )
    def k(dst_hbm, out_hbm, didx_v, ones_v, zbuf_v, acc_sh, sem):
        c = lax.axis_index("c")
        s = lax.axis_index("s")
        wid = c * NS + s
        hot = (1 - jnp.minimum(lax.iota(jnp.int32, 16), 1)).astype(jnp.float32)
        zero = jnp.zeros((16,), jnp.float32)

        def fill_ones(i, _):
            ones_v[i, :] = hot
            return 0

        lax.fori_loop(0, CH, fill_ones, 0)

        def fill_zero(i, _):
            zbuf_v[i, :] = zero
            return 0

        lax.fori_loop(0, zr, fill_zero, 0)
        for t in range(5):
            pltpu.sync_copy(zbuf_v, acc_sh.at[pl.ds(s * rw + t * zr, zr)])
        plsc.subcore_barrier()

        def super_body(g, _):
            base = wid * ew + g * (KDEPTH * CH)
            descs = []
            for j in range(KDEPTH):
                descs.append(pltpu.async_copy(
                    dst_hbm.at[pl.ds(base + j * CH, CH)], didx_v.at[j], sem))
            for j in range(KDEPTH):
                descs[j].wait()
                pltpu.sync_copy(ones_v, acc_sh.at[didx_v.at[j]], add=True)
            return 0

        lax.fori_loop(0, cw // KDEPTH, super_body, 0)
        plsc.subcore_barrier()
        pltpu.sync_copy(
            acc_sh.at[pl.ds(s * rw, rw)], out_hbm.at[c, pl.ds(s * rw, rw)]
        )

    return k(dst1d)


def _sc_segsum(table, src1d, dst1d):
    n, f = table.shape
    e = src1d.shape[0]
    ew = e // NW
    cw = ew // CH
    rw = NP // NS
    zr = rw // 5

    @functools.partial(
        pl.kernel,
        out_type=jax.ShapeDtypeStruct((NC, NP, f), jnp.float32),
        mesh=_mesh(),
        compiler_params=pltpu.CompilerParams(use_tc_tiling_on_sc=False),
        scratch_types=[
            pltpu.VMEM((KDEPTH, CH), jnp.int32),
            pltpu.VMEM((KDEPTH, CH), jnp.int32),
            pltpu.VMEM((KDEPTH, CH, f), jnp.float32),
            pltpu.VMEM((zr, f), jnp.float32),
            pltpu.MemorySpace.VMEM_SHARED((NP, f), jnp.float32),
            pltpu.SemaphoreType.DMA,
            pltpu.SemaphoreType.DMA,
        ],
    )
    def k(table_hbm, src_hbm, dst_hbm, out_hbm,
          sidx_v, didx_v, rows_v, zbuf_v, acc_sh, isem, gsem):
        c = lax.axis_index("c")
        s = lax.axis_index("s")
        wid = c * NS + s
        zero = jnp.zeros((16,), jnp.float32)

        def fill_zero(i, _):
            for j in range(f // 16):
                zbuf_v[i, pl.ds(16 * j, 16)] = zero
            return 0

        lax.fori_loop(0, zr, fill_zero, 0)
        for t in range(5):
            pltpu.sync_copy(zbuf_v, acc_sh.at[pl.ds(s * rw + t * zr, zr)])
        plsc.subcore_barrier()

        def super_body(g, _):
            base = wid * ew + g * (KDEPTH * CH)
            idesc = []
            for j in range(KDEPTH):
                idesc.append(pltpu.async_copy(
                    src_hbm.at[pl.ds(base + j * CH, CH)], sidx_v.at[j], isem))
                idesc.append(pltpu.async_copy(
                    dst_hbm.at[pl.ds(base + j * CH, CH)], didx_v.at[j], isem))
            for dsc in idesc:
                dsc.wait()
            gdesc = []
            for j in range(KDEPTH):
                gdesc.append(pltpu.async_copy(
                    table_hbm.at[sidx_v.at[j]], rows_v.at[j], gsem))
            for j in range(KDEPTH):
                gdesc[j].wait()
                pltpu.sync_copy(
                    rows_v.at[j], acc_sh.at[didx_v.at[j]], add=True)
            return 0

        lax.fori_loop(0, cw // KDEPTH, super_body, 0)
        plsc.subcore_barrier()
        pltpu.sync_copy(
            acc_sh.at[pl.ds(s * rw, rw)], out_hbm.at[c, pl.ds(s * rw, rw)]
        )

    return k(table, src1d, dst1d)


def _tc_call(body, out_shapes, *args):
    return pl.pallas_call(body, out_shape=out_shapes)(*args)


def _k1_body(x_ref, w1_ref, degp_ref, h1p_ref, dinv_ref):
    n = x_ref.shape[0]
    deg = degp_ref[0, :n, 0:1] + degp_ref[1, :n, 0:1] + 1.0
    dinv = lax.rsqrt(deg)
    h = jnp.dot(x_ref[...], w1_ref[...], preferred_element_type=jnp.float32)
    h1p_ref[...] = h * dinv
    dinv_ref[...] = dinv


def _k2_body(a_ref, h1p_ref, dinv_ref, b1_ref, w2_ref, h2p_ref):
    n = h1p_ref.shape[0]
    dinv = dinv_ref[...]
    agg = a_ref[0, :n] + a_ref[1, :n]
    h1 = jnp.maximum(dinv * (agg + h1p_ref[...]) + b1_ref[...], 0.0)
    h2p_ref[...] = jnp.dot(
        h1, w2_ref[...], preferred_element_type=jnp.float32
    ) * dinv


def _k3_body(a_ref, h2p_ref, dinv_ref, b2_ref, batch_ref, w3_ref, h3p_ref):
    n = h2p_ref.shape[0]
    dinv = dinv_ref[...]
    agg = a_ref[0, :n] + a_ref[1, :n]
    h2 = dinv * (agg + h2p_ref[...]) + b2_ref[...]
    gids = lax.broadcasted_iota(jnp.int32, (n, 8), 1)
    onehot = (batch_ref[...] == gids).astype(jnp.float32)
    counts = jnp.sum(onehot, axis=0, keepdims=True)
    zsum = lax.dot_general(
        onehot, h2, (((0,), (0,)), ((), ())),
        preferred_element_type=jnp.float32,
    )
    z = zsum / jnp.maximum(counts, 1.0).T
    u = jnp.dot(z, w3_ref[...], preferred_element_type=jnp.float32)
    h3p_ref[...] = jnp.dot(
        onehot, u, preferred_element_type=jnp.float32
    ) * dinv


def _k4_body(a_ref, h3p_ref, dinv_ref, b3_ref, h4p_ref):
    n = h3p_ref.shape[0]
    dinv = dinv_ref[...]
    agg = a_ref[0, :n] + a_ref[1, :n]
    h3 = jnp.maximum(dinv * (agg + h3p_ref[...]) + b3_ref[...], 0.0)
    h4p_ref[...] = h3 * dinv


def _k5_body(a_ref, h4p_ref, dinv_ref, b4_ref, w4_ref, out_ref):
    n = h4p_ref.shape[0]
    dinv = dinv_ref[...]
    agg = a_ref[0, :n] + a_ref[1, :n]
    ah = dinv * (agg + h4p_ref[...])
    out_ref[...] = jnp.dot(
        ah, w4_ref[...], preferred_element_type=jnp.float32
    ) + b4_ref[...]


def kernel(x, edge_index, batch, W1, b1, W2, b2, W3, b3, W4, b4):
    n, d = x.shape
    h, l = W1.shape[1], W2.shape[1]

    src1d = edge_index[0]
    dst1d = edge_index[1]
    batch2d = batch.reshape(n, 1)

    degp = _sc_degree(dst1d)

    h1p, dinv = _tc_call(
        _k1_body,
        (jax.ShapeDtypeStruct((n, h), jnp.float32),
         jax.ShapeDtypeStruct((n, 1), jnp.float32)),
        x, W1, degp,
    )
    agg1 = _sc_segsum(h1p, src1d, dst1d)
    h2p = _tc_call(
        _k2_body, jax.ShapeDtypeStruct((n, l), jnp.float32),
        agg1, h1p, dinv, b1, W2,
    )
    agg2 = _sc_segsum(h2p, src1d, dst1d)
    h3p = _tc_call(
        _k3_body, jax.ShapeDtypeStruct((n, h), jnp.float32),
        agg2, h2p, dinv, b2, batch2d, W3,
    )
    agg3 = _sc_segsum(h3p, src1d, dst1d)
    h4p = _tc_call(
        _k4_body, jax.ShapeDtypeStruct((n, h), jnp.float32),
        agg3, h3p, dinv, b3,
    )
    agg4 = _sc_segsum(h4p, src1d, dst1d)
    out = _tc_call(
        _k5_body, jax.ShapeDtypeStruct((n, d), jnp.float32),
        agg4, h4p, dinv, b4, W4,
    )
    return out


# speedup vs baseline: 34.6550x; 1.1633x over previous
import functools

import jax
import jax.numpy as jnp
from jax import lax
from jax.experimental import pallas as pl
from jax.experimental.pallas import tpu as pltpu
from jax.experimental.pallas import tpu_sc as plsc

NC, NS = 2, 16
NW = NC * NS
CH = 80
KDEPTH = 5
NP = 10240

_mesh = functools.partial(
    plsc.VectorSubcoreMesh, core_axis_name="c", subcore_axis_name="s"
)


def _sc_degree(dst1d):
    e = dst1d.shape[0]
    ew = e // NW
    cw = ew // CH
    rw = NP // NS
    zr = rw // 5

    @functools.partial(
        pl.kernel,
        out_type=jax.ShapeDtypeStruct((NC, NP, 16), jnp.float32),
        mesh=_mesh(),
        compiler_params=pltpu.CompilerParams(use_tc_tiling_on_sc=False),
        scratch_types=[
            pltpu.VMEM((KDEPTH, CH), jnp.int32),
            pltpu.VMEM((CH, 16), jnp.float32),
            pltpu.VMEM((zr, 16), jnp.float32),
            pltpu.MemorySpace.VMEM_SHARED((NP, 16), jnp.float32),
            pltpu.SemaphoreType.DMA,
        ],
    )
    def k(dst_hbm, out_hbm, didx_v, ones_v, zbuf_v, acc_sh, sem):
        c = lax.axis_index("c")
        s = lax.axis_index("s")
        wid = c * NS + s
        hot = (1 - jnp.minimum(lax.iota(jnp.int32, 16), 1)).astype(jnp.float32)
        zero = jnp.zeros((16,), jnp.float32)

        def fill_ones(i, _):
            ones_v[i, :] = hot
            return 0

        lax.fori_loop(0, CH, fill_ones, 0)

        def fill_zero(i, _):
            zbuf_v[i, :] = zero
            return 0

        lax.fori_loop(0, zr, fill_zero, 0)
        for t in range(5):
            pltpu.sync_copy(zbuf_v, acc_sh.at[pl.ds(s * rw + t * zr, zr)])
        plsc.subcore_barrier()

        def super_body(g, _):
            base = wid * ew + g * (KDEPTH * CH)
            descs = []
            for j in range(KDEPTH):
                descs.append(pltpu.async_copy(
                    dst_hbm.at[pl.ds(base + j * CH, CH)], didx_v.at[j], sem))
            for j in range(KDEPTH):
                descs[j].wait()
                pltpu.sync_copy(ones_v, acc_sh.at[didx_v.at[j]], add=True)
            return 0

        lax.fori_loop(0, cw // KDEPTH, super_body, 0)
        plsc.subcore_barrier()
        pltpu.sync_copy(
            acc_sh.at[pl.ds(s * rw, rw)], out_hbm.at[c, pl.ds(s * rw, rw)]
        )

    return k(dst1d)


def _sc_segsum(table, src1d, dst1d):
    n, f = table.shape
    e = src1d.shape[0]
    ew = e // NW
    cw = ew // CH
    rw = NP // NS
    zr = rw // 5

    nsup = cw // KDEPTH

    @functools.partial(
        pl.kernel,
        out_type=jax.ShapeDtypeStruct((NC, NP, f), jnp.float32),
        mesh=_mesh(),
        compiler_params=pltpu.CompilerParams(use_tc_tiling_on_sc=False),
        scratch_types=[
            pltpu.VMEM((2, KDEPTH, CH), jnp.int32),
            pltpu.VMEM((2, KDEPTH, CH), jnp.int32),
            pltpu.VMEM((2, KDEPTH, CH, f), jnp.float32),
            pltpu.VMEM((zr, f), jnp.float32),
            pltpu.MemorySpace.VMEM_SHARED((NP, f), jnp.float32),
            pltpu.SemaphoreType.DMA,
            pltpu.SemaphoreType.DMA,
            pltpu.SemaphoreType.DMA,
            pltpu.SemaphoreType.DMA,
        ],
    )
    def k(table_hbm, src_hbm, dst_hbm, out_hbm,
          sidx_v, didx_v, rows_v, zbuf_v, acc_sh, isem, gsem, ssem, zsem):
        c = lax.axis_index("c")
        s = lax.axis_index("s")
        wid = c * NS + s
        zero = jnp.zeros((16,), jnp.float32)

        def fill_zero(i, _):
            for j in range(f // 16):
                zbuf_v[i, pl.ds(16 * j, 16)] = zero
            return 0

        lax.fori_loop(0, zr, fill_zero, 0)
        zdesc = [
            pltpu.async_copy(
                zbuf_v, acc_sh.at[pl.ds(s * rw + t * zr, zr)], zsem)
            for t in range(5)
        ]
        for dsc in zdesc:
            dsc.wait()
        plsc.subcore_barrier()

        def fire_idx(g, sel):
            base = wid * ew + g * (KDEPTH * CH)
            out = []
            for j in range(KDEPTH):
                out.append(pltpu.async_copy(
                    src_hbm.at[pl.ds(base + j * CH, CH)],
                    sidx_v.at[sel, j], isem))
                out.append(pltpu.async_copy(
                    dst_hbm.at[pl.ds(base + j * CH, CH)],
                    didx_v.at[sel, j], isem))
            return out

        for dsc in fire_idx(0, 0):
            dsc.wait()

        def super_body(g, _):
            sel = g % 2
            oth = 1 - sel

            @pl.when(g > 0)
            def _():
                for j in range(KDEPTH):
                    pltpu.make_async_copy(
                        rows_v.at[oth, j],
                        acc_sh.at[didx_v.at[oth, j]], ssem).wait()

            gdesc = []
            for j in range(KDEPTH):
                gdesc.append(pltpu.async_copy(
                    table_hbm.at[sidx_v.at[sel, j]], rows_v.at[sel, j], gsem))

            @pl.when(g + 1 < nsup)
            def _():
                fire_idx(g + 1, oth)

            for j in range(KDEPTH):
                gdesc[j].wait()
                pltpu.async_copy(
                    rows_v.at[sel, j], acc_sh.at[didx_v.at[sel, j]],
                    ssem, add=True)

            @pl.when(g + 1 < nsup)
            def _():
                base = wid * ew + (g + 1) * (KDEPTH * CH)
                for j in range(KDEPTH):
                    pltpu.make_async_copy(
                        src_hbm.at[pl.ds(base + j * CH, CH)],
                        sidx_v.at[oth, j], isem).wait()
                    pltpu.make_async_copy(
                        dst_hbm.at[pl.ds(base + j * CH, CH)],
                        didx_v.at[oth, j], isem).wait()
            return 0

        lax.fori_loop(0, nsup, super_body, 0)
        lastsel = (nsup - 1) % 2
        for j in range(KDEPTH):
            pltpu.make_async_copy(
                rows_v.at[lastsel, j],
                acc_sh.at[didx_v.at[lastsel, j]], ssem).wait()
        plsc.subcore_barrier()
        pltpu.sync_copy(
            acc_sh.at[pl.ds(s * rw, rw)], out_hbm.at[c, pl.ds(s * rw, rw)]
        )

    return k(table, src1d, dst1d)


def _tc_call(body, out_shapes, *args):
    return pl.pallas_call(body, out_shape=out_shapes)(*args)


def _k1_body(x_ref, w1_ref, degp_ref, h1p_ref, dinv_ref):
    n = x_ref.shape[0]
    deg = degp_ref[0, :n, 0:1] + degp_ref[1, :n, 0:1] + 1.0
    dinv = lax.rsqrt(deg)
    h = jnp.dot(x_ref[...], w1_ref[...], preferred_element_type=jnp.float32)
    h1p_ref[...] = h * dinv
    dinv_ref[...] = dinv


def _k2_body(a_ref, h1p_ref, dinv_ref, b1_ref, w2_ref, h2p_ref):
    n = h1p_ref.shape[0]
    dinv = dinv_ref[...]
    agg = a_ref[0, :n] + a_ref[1, :n]
    h1 = jnp.maximum(dinv * (agg + h1p_ref[...]) + b1_ref[...], 0.0)
    h2p_ref[...] = jnp.dot(
        h1, w2_ref[...], preferred_element_type=jnp.float32
    ) * dinv


def _k3_body(a_ref, h2p_ref, dinv_ref, b2_ref, batch_ref, w3_ref, h3p_ref):
    n = h2p_ref.shape[0]
    dinv = dinv_ref[...]
    agg = a_ref[0, :n] + a_ref[1, :n]
    h2 = dinv * (agg + h2p_ref[...]) + b2_ref[...]
    gids = lax.broadcasted_iota(jnp.int32, (n, 8), 1)
    onehot = (batch_ref[...] == gids).astype(jnp.float32)
    counts = jnp.sum(onehot, axis=0, keepdims=True)
    zsum = lax.dot_general(
        onehot, h2, (((0,), (0,)), ((), ())),
        preferred_element_type=jnp.float32,
    )
    z = zsum / jnp.maximum(counts, 1.0).T
    u = jnp.dot(z, w3_ref[...], preferred_element_type=jnp.float32)
    h3p_ref[...] = jnp.dot(
        onehot, u, preferred_element_type=jnp.float32
    ) * dinv


def _k4_body(a_ref, h3p_ref, dinv_ref, b3_ref, h4p_ref):
    n = h3p_ref.shape[0]
    dinv = dinv_ref[...]
    agg = a_ref[0, :n] + a_ref[1, :n]
    h3 = jnp.maximum(dinv * (agg + h3p_ref[...]) + b3_ref[...], 0.0)
    h4p_ref[...] = h3 * dinv


def _k5_body(a_ref, h4p_ref, dinv_ref, b4_ref, w4_ref, out_ref):
    n = h4p_ref.shape[0]
    dinv = dinv_ref[...]
    agg = a_ref[0, :n] + a_ref[1, :n]
    ah = dinv * (agg + h4p_ref[...])
    out_ref[...] = jnp.dot(
        ah, w4_ref[...], preferred_element_type=jnp.float32
    ) + b4_ref[...]


def kernel(x, edge_index, batch, W1, b1, W2, b2, W3, b3, W4, b4):
    n, d = x.shape
    h, l = W1.shape[1], W2.shape[1]

    src1d = edge_index[0]
    dst1d = edge_index[1]
    batch2d = batch.reshape(n, 1)

    degp = _sc_degree(dst1d)

    h1p, dinv = _tc_call(
        _k1_body,
        (jax.ShapeDtypeStruct((n, h), jnp.float32),
         jax.ShapeDtypeStruct((n, 1), jnp.float32)),
        x, W1, degp,
    )
    agg1 = _sc_segsum(h1p, src1d, dst1d)
    h2p = _tc_call(
        _k2_body, jax.ShapeDtypeStruct((n, l), jnp.float32),
        agg1, h1p, dinv, b1, W2,
    )
    agg2 = _sc_segsum(h2p, src1d, dst1d)
    h3p = _tc_call(
        _k3_body, jax.ShapeDtypeStruct((n, h), jnp.float32),
        agg2, h2p, dinv, b2, batch2d, W3,
    )
    agg3 = _sc_segsum(h3p, src1d, dst1d)
    h4p = _tc_call(
        _k4_body, jax.ShapeDtypeStruct((n, h), jnp.float32),
        agg3, h3p, dinv, b3,
    )
    agg4 = _sc_segsum(h4p, src1d, dst1d)
    out = _tc_call(
        _k5_body, jax.ShapeDtypeStruct((n, d), jnp.float32),
        agg4, h4p, dinv, b4, W4,
    )
    return out


# speedup vs baseline: 41.5861x; 1.2000x over previous
import functools

import jax
import jax.numpy as jnp
from jax import lax
from jax.experimental import pallas as pl
from jax.experimental.pallas import tpu as pltpu
from jax.experimental.pallas import tpu_sc as plsc

NC, NS = 2, 16
NW = NC * NS
CH = 80
KDEPTH = 5
NP = 10240

_mesh = functools.partial(
    plsc.VectorSubcoreMesh, core_axis_name="c", subcore_axis_name="s"
)


def _sc_degree(dst1d):
    e = dst1d.shape[0]
    ew = e // NW
    cw = ew // CH
    rw = NP // NS
    zr = rw // 5

    @functools.partial(
        pl.kernel,
        out_type=jax.ShapeDtypeStruct((NC, NP, 16), jnp.float32),
        mesh=_mesh(),
        compiler_params=pltpu.CompilerParams(
            use_tc_tiling_on_sc=False, needs_layout_passes=False),
        scratch_types=[
            pltpu.VMEM((KDEPTH, CH), jnp.int32),
            pltpu.VMEM((CH, 16), jnp.float32),
            pltpu.VMEM((zr, 16), jnp.float32),
            pltpu.MemorySpace.VMEM_SHARED((NP, 16), jnp.float32),
            pltpu.SemaphoreType.DMA,
        ],
    )
    def k(dst_hbm, out_hbm, didx_v, ones_v, zbuf_v, acc_sh, sem):
        c = lax.axis_index("c")
        s = lax.axis_index("s")
        wid = c * NS + s
        hot = (1 - jnp.minimum(lax.iota(jnp.int32, 16), 1)).astype(jnp.float32)
        zero = jnp.zeros((16,), jnp.float32)

        def fill_ones(i, _):
            ones_v[i, :] = hot
            return 0

        lax.fori_loop(0, CH, fill_ones, 0)

        def fill_zero(i, _):
            zbuf_v[i, :] = zero
            return 0

        lax.fori_loop(0, zr, fill_zero, 0)
        for t in range(5):
            pltpu.sync_copy(zbuf_v, acc_sh.at[pl.ds(s * rw + t * zr, zr)])
        plsc.subcore_barrier()

        def super_body(g, _):
            base = wid * ew + g * (KDEPTH * CH)
            descs = []
            for j in range(KDEPTH):
                descs.append(pltpu.async_copy(
                    dst_hbm.at[pl.ds(base + j * CH, CH)], didx_v.at[j], sem))
            for j in range(KDEPTH):
                descs[j].wait()
                pltpu.sync_copy(ones_v, acc_sh.at[didx_v.at[j]], add=True)
            return 0

        lax.fori_loop(0, cw // KDEPTH, super_body, 0)
        plsc.subcore_barrier()
        pltpu.sync_copy(
            acc_sh.at[pl.ds(s * rw, rw)], out_hbm.at[c, pl.ds(s * rw, rw)]
        )

    return k(dst1d)


def _sc_segsum(table, src1d, dst1d):
    n, f = table.shape
    e = src1d.shape[0]
    ew = e // NW
    cw = ew // CH
    rw = NP // NS
    zr = rw // 5

    nsup = cw // KDEPTH

    @functools.partial(
        pl.kernel,
        out_type=jax.ShapeDtypeStruct((NC, NP, f), jnp.float32),
        mesh=_mesh(),
        compiler_params=pltpu.CompilerParams(
            use_tc_tiling_on_sc=False, needs_layout_passes=False),
        scratch_types=[
            pltpu.VMEM((2, KDEPTH, CH), jnp.int32),
            pltpu.VMEM((2, KDEPTH, CH), jnp.int32),
            pltpu.VMEM((2, KDEPTH, CH, f), jnp.float32),
            pltpu.VMEM((zr, f), jnp.float32),
            pltpu.MemorySpace.VMEM_SHARED((NP, f), jnp.float32),
            pltpu.SemaphoreType.DMA,
            pltpu.SemaphoreType.DMA,
            pltpu.SemaphoreType.DMA,
            pltpu.SemaphoreType.DMA,
        ],
    )
    def k(table_hbm, src_hbm, dst_hbm, out_hbm,
          sidx_v, didx_v, rows_v, zbuf_v, acc_sh, isem, gsem, ssem, zsem):
        c = lax.axis_index("c")
        s = lax.axis_index("s")
        wid = c * NS + s
        zero = jnp.zeros((16,), jnp.float32)

        def fire_idx(g, sel):
            base = wid * ew + g * (KDEPTH * CH)
            out = []
            for j in range(KDEPTH):
                out.append(pltpu.async_copy(
                    src_hbm.at[pl.ds(base + j * CH, CH)],
                    sidx_v.at[sel, j], isem))
                out.append(pltpu.async_copy(
                    dst_hbm.at[pl.ds(base + j * CH, CH)],
                    didx_v.at[sel, j], isem))
            return out

        idesc0 = fire_idx(0, 0)

        def fill_zero(i, _):
            for j in range(f // 16):
                zbuf_v[i, pl.ds(16 * j, 16)] = zero
            return 0

        lax.fori_loop(0, zr, fill_zero, 0)
        zdesc = [
            pltpu.async_copy(
                zbuf_v, acc_sh.at[pl.ds(s * rw + t * zr, zr)], zsem)
            for t in range(5)
        ]
        for dsc in zdesc:
            dsc.wait()
        plsc.subcore_barrier()
        for dsc in idesc0:
            dsc.wait()

        def super_body(g, _):
            sel = g % 2
            oth = 1 - sel

            @pl.when(g > 0)
            def _():
                for j in range(KDEPTH):
                    pltpu.make_async_copy(
                        rows_v.at[oth, j],
                        acc_sh.at[didx_v.at[oth, j]], ssem).wait()

            gdesc = []
            for j in range(KDEPTH):
                gdesc.append(pltpu.async_copy(
                    table_hbm.at[sidx_v.at[sel, j]], rows_v.at[sel, j], gsem))

            @pl.when(g + 1 < nsup)
            def _():
                fire_idx(g + 1, oth)

            for j in range(KDEPTH):
                gdesc[j].wait()
                pltpu.async_copy(
                    rows_v.at[sel, j], acc_sh.at[didx_v.at[sel, j]],
                    ssem, add=True)

            @pl.when(g + 1 < nsup)
            def _():
                base = wid * ew + (g + 1) * (KDEPTH * CH)
                for j in range(KDEPTH):
                    pltpu.make_async_copy(
                        src_hbm.at[pl.ds(base + j * CH, CH)],
                        sidx_v.at[oth, j], isem).wait()
                    pltpu.make_async_copy(
                        dst_hbm.at[pl.ds(base + j * CH, CH)],
                        didx_v.at[oth, j], isem).wait()
            return 0

        lax.fori_loop(0, nsup, super_body, 0)
        lastsel = (nsup - 1) % 2
        for j in range(KDEPTH):
            pltpu.make_async_copy(
                rows_v.at[lastsel, j],
                acc_sh.at[didx_v.at[lastsel, j]], ssem).wait()
        plsc.subcore_barrier()
        pltpu.sync_copy(
            acc_sh.at[pl.ds(s * rw, rw)], out_hbm.at[c, pl.ds(s * rw, rw)]
        )

    return k(table, src1d, dst1d)


def _sc_segsum_m(table, src1d, dst1d, batch1d, dinv1d, nb):
    n, f = table.shape
    e = src1d.shape[0]
    ew = e // NW
    cw = ew // CH
    rw = NP // NS
    zr = rw // 5
    nsup = cw // KDEPTH
    gr = CH // 16

    @functools.partial(
        pl.kernel,
        out_type=(jax.ShapeDtypeStruct((NC, NP, f), jnp.float32),
                  jax.ShapeDtypeStruct((NC, NP, nb), jnp.float32)),
        mesh=_mesh(),
        compiler_params=pltpu.CompilerParams(
            use_tc_tiling_on_sc=False, needs_layout_passes=False),
        scratch_types=[
            pltpu.VMEM((2, KDEPTH, CH), jnp.int32),
            pltpu.VMEM((2, KDEPTH, CH), jnp.int32),
            pltpu.VMEM((2, KDEPTH, CH, f), jnp.float32),
            pltpu.VMEM((2, KDEPTH, CH, nb), jnp.float32),
            pltpu.VMEM((2, KDEPTH, CH), jnp.int32),
            pltpu.VMEM((zr, f), jnp.float32),
            pltpu.VMEM((n,), jnp.int32),
            pltpu.VMEM((n,), jnp.float32),
            pltpu.MemorySpace.VMEM_SHARED((NP, f), jnp.float32),
            pltpu.MemorySpace.VMEM_SHARED((NP, nb), jnp.float32),
            pltpu.SemaphoreType.DMA,
            pltpu.SemaphoreType.DMA,
            pltpu.SemaphoreType.DMA,
            pltpu.SemaphoreType.DMA,
            pltpu.SemaphoreType.DMA,
            pltpu.SemaphoreType.DMA,
        ],
    )
    def k(table_hbm, src_hbm, dst_hbm, batch_hbm, dinv_hbm,
          out_hbm, mout_hbm,
          sidx_v, didx_v, rows_v, mrows_v, bcol_v, zbuf_v, batch_t, dinv_t,
          acc_sh, macc_sh, isem, gsem, ssem, zsem, msem, tsem):
        c = lax.axis_index("c")
        s = lax.axis_index("s")
        wid = c * NS + s
        zero = jnp.zeros((16,), jnp.float32)
        iota16 = lax.iota(jnp.int32, 16)

        tdesc = [pltpu.async_copy(batch_hbm, batch_t, tsem),
                 pltpu.async_copy(dinv_hbm, dinv_t, tsem)]

        def fire_idx(g, sel):
            base = wid * ew + g * (KDEPTH * CH)
            out = []
            for j in range(KDEPTH):
                out.append(pltpu.async_copy(
                    src_hbm.at[pl.ds(base + j * CH, CH)],
                    sidx_v.at[sel, j], isem))
                out.append(pltpu.async_copy(
                    dst_hbm.at[pl.ds(base + j * CH, CH)],
                    didx_v.at[sel, j], isem))
            return out

        idesc0 = fire_idx(0, 0)

        nbv = jnp.full((16,), nb, jnp.int32)

        def zero_mrows(a, j):
            for kk in range(CH * nb // 16):
                rowv = lax.div(kk * 16 + iota16, nbv)
                colv = lax.rem(kk * 16 + iota16, nbv)
                plsc.store_scatter(mrows_v.at[a, j], [rowv, colv], zero)

        zero_mrows(0, 0)
        zdesc = [
            pltpu.async_copy(
                mrows_v.at[0, 0],
                macc_sh.at[pl.ds(s * rw + t * CH, CH)], zsem)
            for t in range(rw // CH)
        ]
        for a in range(2):
            for j in range(KDEPTH):
                if not (a == 0 and j == 0):
                    zero_mrows(a, j)

        def fill_zero(i, _):
            for j in range(f // 16):
                zbuf_v[i, pl.ds(16 * j, 16)] = zero
            return 0

        lax.fori_loop(0, zr, fill_zero, 0)
        zdesc += [
            pltpu.async_copy(
                zbuf_v, acc_sh.at[pl.ds(s * rw + t * zr, zr)], zsem)
            for t in range(5)
        ]
        for dsc in zdesc:
            dsc.wait()
        plsc.subcore_barrier()
        for dsc in idesc0 + tdesc:
            dsc.wait()

        def super_body(g, _):
            sel = g % 2
            oth = 1 - sel

            @pl.when(g > 0)
            def _():
                for j in range(KDEPTH):
                    pltpu.make_async_copy(
                        rows_v.at[oth, j],
                        acc_sh.at[didx_v.at[oth, j]], ssem).wait()
                    pltpu.make_async_copy(
                        mrows_v.at[oth, j],
                        macc_sh.at[didx_v.at[oth, j]], msem).wait()
                    for i in range(gr):
                        b16 = bcol_v[oth, j, pl.ds(i * 16, 16)]
                        plsc.store_scatter(
                            mrows_v.at[oth, j], [i * 16 + iota16, b16], zero)

            gdesc = []
            for j in range(KDEPTH):
                gdesc.append(pltpu.async_copy(
                    table_hbm.at[sidx_v.at[sel, j]], rows_v.at[sel, j], gsem))

            @pl.when(g + 1 < nsup)
            def _():
                fire_idx(g + 1, oth)

            for j in range(KDEPTH):
                for i in range(gr):
                    srcv = sidx_v[sel, j, pl.ds(i * 16, 16)]
                    b16 = plsc.load_gather(batch_t, [srcv])
                    d16 = plsc.load_gather(dinv_t, [srcv])
                    plsc.store_scatter(
                        mrows_v.at[sel, j], [i * 16 + iota16, b16], d16)
                    bcol_v[sel, j, pl.ds(i * 16, 16)] = b16
                pltpu.async_copy(
                    mrows_v.at[sel, j], macc_sh.at[didx_v.at[sel, j]],
                    msem, add=True)

            for j in range(KDEPTH):
                gdesc[j].wait()
                pltpu.async_copy(
                    rows_v.at[sel, j], acc_sh.at[didx_v.at[sel, j]],
                    ssem, add=True)

            @pl.when(g + 1 < nsup)
            def _():
                base = wid * ew + (g + 1) * (KDEPTH * CH)
                for j in range(KDEPTH):
                    pltpu.make_async_copy(
                        src_hbm.at[pl.ds(base + j * CH, CH)],
                        sidx_v.at[oth, j], isem).wait()
                    pltpu.make_async_copy(
                        dst_hbm.at[pl.ds(base + j * CH, CH)],
                        didx_v.at[oth, j], isem).wait()
            return 0

        lax.fori_loop(0, nsup, super_body, 0)
        lastsel = (nsup - 1) % 2
        for j in range(KDEPTH):
            pltpu.make_async_copy(
                rows_v.at[lastsel, j],
                acc_sh.at[didx_v.at[lastsel, j]], ssem).wait()
            pltpu.make_async_copy(
                mrows_v.at[lastsel, j],
                macc_sh.at[didx_v.at[lastsel, j]], msem).wait()
        plsc.subcore_barrier()
        pltpu.sync_copy(
            acc_sh.at[pl.ds(s * rw, rw)], out_hbm.at[c, pl.ds(s * rw, rw)]
        )
        pltpu.sync_copy(
            macc_sh.at[pl.ds(s * rw, rw)], mout_hbm.at[c, pl.ds(s * rw, rw)]
        )

    return k(table, src1d, dst1d, batch1d, dinv1d)


def _tc_call(body, out_shapes, *args):
    return pl.pallas_call(body, out_shape=out_shapes)(*args)


def _k1_body(x_ref, w1_ref, degp_ref, h1p_ref, dinv_ref):
    n = x_ref.shape[0]
    deg = degp_ref[0, :n, 0:1] + degp_ref[1, :n, 0:1] + 1.0
    dinv = lax.rsqrt(deg)
    h = jnp.dot(x_ref[...], w1_ref[...], preferred_element_type=jnp.float32)
    h1p_ref[...] = h * dinv
    dinv_ref[...] = dinv


def _k2_body(a_ref, h1p_ref, dinv_ref, b1_ref, w2_ref, h2p_ref):
    n = h1p_ref.shape[0]
    dinv = dinv_ref[...]
    agg = a_ref[0, :n] + a_ref[1, :n]
    h1 = jnp.maximum(dinv * (agg + h1p_ref[...]) + b1_ref[...], 0.0)
    h2p_ref[...] = jnp.dot(
        h1, w2_ref[...], preferred_element_type=jnp.float32
    ) * dinv


def _k34_body(a_ref, h2p_ref, dinv_ref, b2_ref, batch_ref, w3_ref,
              mp_ref, b3_ref, h4p_ref):
    n = h2p_ref.shape[0]
    dinv = dinv_ref[...]
    agg = a_ref[0, :n] + a_ref[1, :n]
    h2 = dinv * (agg + h2p_ref[...]) + b2_ref[...]
    gids = lax.broadcasted_iota(jnp.int32, (n, 8), 1)
    onehot = (batch_ref[...] == gids).astype(jnp.float32)
    counts = jnp.sum(onehot, axis=0, keepdims=True)
    zsum = lax.dot_general(
        onehot, h2, (((0,), (0,)), ((), ())),
        preferred_element_type=jnp.float32,
    )
    z = zsum / jnp.maximum(counts, 1.0).T
    u = jnp.dot(z, w3_ref[...], preferred_element_type=jnp.float32)
    h3p = jnp.dot(onehot, u, preferred_element_type=jnp.float32) * dinv
    m = mp_ref[0, :n] + mp_ref[1, :n]
    agg3 = jnp.dot(m, u, preferred_element_type=jnp.float32)
    h3 = jnp.maximum(dinv * (agg3 + h3p) + b3_ref[...], 0.0)
    h4p_ref[...] = h3 * dinv


def _k5_body(a_ref, h4p_ref, dinv_ref, b4_ref, w4_ref, out_ref):
    n = h4p_ref.shape[0]
    dinv = dinv_ref[...]
    agg = a_ref[0, :n] + a_ref[1, :n]
    ah = dinv * (agg + h4p_ref[...])
    out_ref[...] = jnp.dot(
        ah, w4_ref[...], preferred_element_type=jnp.float32
    ) + b4_ref[...]


def kernel(x, edge_index, batch, W1, b1, W2, b2, W3, b3, W4, b4):
    n, d = x.shape
    h, l = W1.shape[1], W2.shape[1]

    src1d = edge_index[0]
    dst1d = edge_index[1]
    batch2d = batch.reshape(n, 1)

    degp = _sc_degree(dst1d)

    h1p, dinv = _tc_call(
        _k1_body,
        (jax.ShapeDtypeStruct((n, h), jnp.float32),
         jax.ShapeDtypeStruct((n, 1), jnp.float32)),
        x, W1, degp,
    )
    agg1 = _sc_segsum(h1p, src1d, dst1d)
    h2p = _tc_call(
        _k2_body, jax.ShapeDtypeStruct((n, l), jnp.float32),
        agg1, h1p, dinv, b1, W2,
    )
    agg2, mparts = _sc_segsum_m(
        h2p, src1d, dst1d, batch, dinv.reshape(n), 8)
    h4p = _tc_call(
        _k34_body, jax.ShapeDtypeStruct((n, h), jnp.float32),
        agg2, h2p, dinv, b2, batch2d, W3, mparts, b3,
    )
    agg4 = _sc_segsum(h4p, src1d, dst1d)
    out = _tc_call(
        _k5_body, jax.ShapeDtypeStruct((n, d), jnp.float32),
        agg4, h4p, dinv, b4, W4,
    )
    return out


# speedup vs baseline: 43.1284x; 1.0371x over previous
import functools

import jax
import jax.numpy as jnp
from jax import lax
from jax.experimental import pallas as pl
from jax.experimental.pallas import tpu as pltpu
from jax.experimental.pallas import tpu_sc as plsc

NC, NS = 2, 16
NW = NC * NS
CH = 80
CHM = 80
KDEPTH = 5
NP = 10240

_mesh = functools.partial(
    plsc.VectorSubcoreMesh, core_axis_name="c", subcore_axis_name="s"
)


def _sc_degree(dst1d):
    e = dst1d.shape[0]
    ew = e // NW
    cw = ew // CH
    rw = NP // NS
    zr = rw // 5

    nsup = cw // KDEPTH

    @functools.partial(
        pl.kernel,
        out_type=jax.ShapeDtypeStruct((NC, NP, 16), jnp.float32),
        mesh=_mesh(),
        compiler_params=pltpu.CompilerParams(
            use_tc_tiling_on_sc=False, needs_layout_passes=False),
        scratch_types=[
            pltpu.VMEM((2, KDEPTH, CH), jnp.int32),
            pltpu.VMEM((CH, 16), jnp.float32),
            pltpu.VMEM((zr, 16), jnp.float32),
            pltpu.MemorySpace.VMEM_SHARED((NP, 16), jnp.float32),
            pltpu.SemaphoreType.DMA,
            pltpu.SemaphoreType.DMA,
        ],
    )
    def k(dst_hbm, out_hbm, didx_v, ones_v, zbuf_v, acc_sh, isem, ssem):
        c = lax.axis_index("c")
        s = lax.axis_index("s")
        wid = c * NS + s
        hot = (1 - jnp.minimum(lax.iota(jnp.int32, 16), 1)).astype(jnp.float32)
        zero = jnp.zeros((16,), jnp.float32)

        def fire_idx(g, sel):
            base = wid * ew + g * (KDEPTH * CH)
            return [pltpu.async_copy(
                dst_hbm.at[pl.ds(base + j * CH, CH)],
                didx_v.at[sel, j], isem) for j in range(KDEPTH)]

        idesc0 = fire_idx(0, 0)

        def fill_ones(i, _):
            ones_v[i, :] = hot
            return 0

        lax.fori_loop(0, CH, fill_ones, 0)

        def fill_zero(i, _):
            zbuf_v[i, :] = zero
            return 0

        lax.fori_loop(0, zr, fill_zero, 0)
        for t in range(5):
            pltpu.sync_copy(zbuf_v, acc_sh.at[pl.ds(s * rw + t * zr, zr)])
        plsc.subcore_barrier()
        for dsc in idesc0:
            dsc.wait()

        def super_body(g, _):
            sel = g % 2
            oth = 1 - sel

            @pl.when(g > 0)
            def _():
                for j in range(KDEPTH):
                    pltpu.make_async_copy(
                        ones_v, acc_sh.at[didx_v.at[oth, j]], ssem).wait()

            for j in range(KDEPTH):
                pltpu.async_copy(
                    ones_v, acc_sh.at[didx_v.at[sel, j]], ssem, add=True)

            @pl.when(g + 1 < nsup)
            def _():
                fire_idx(g + 1, oth)
                base = wid * ew + (g + 1) * (KDEPTH * CH)
                for j in range(KDEPTH):
                    pltpu.make_async_copy(
                        dst_hbm.at[pl.ds(base + j * CH, CH)],
                        didx_v.at[oth, j], isem).wait()
            return 0

        lax.fori_loop(0, nsup, super_body, 0)
        lastsel = (nsup - 1) % 2
        for j in range(KDEPTH):
            pltpu.make_async_copy(
                ones_v, acc_sh.at[didx_v.at[lastsel, j]], ssem).wait()
        plsc.subcore_barrier()
        pltpu.sync_copy(
            acc_sh.at[pl.ds(s * rw, rw)], out_hbm.at[c, pl.ds(s * rw, rw)]
        )

    return k(dst1d)


def _sc_segsum(table, src1d, dst1d):
    n, f = table.shape
    e = src1d.shape[0]
    ew = e // NW
    cw = ew // CH
    rw = NP // NS
    zr = rw // 5

    nsup = cw // KDEPTH

    @functools.partial(
        pl.kernel,
        out_type=jax.ShapeDtypeStruct((NC, NP, f), jnp.float32),
        mesh=_mesh(),
        compiler_params=pltpu.CompilerParams(
            use_tc_tiling_on_sc=False, needs_layout_passes=False),
        scratch_types=[
            pltpu.VMEM((2, KDEPTH, CH), jnp.int32),
            pltpu.VMEM((2, KDEPTH, CH), jnp.int32),
            pltpu.VMEM((2, KDEPTH, CH, f), jnp.float32),
            pltpu.VMEM((zr, f), jnp.float32),
            pltpu.MemorySpace.VMEM_SHARED((NP, f), jnp.float32),
            pltpu.SemaphoreType.DMA,
            pltpu.SemaphoreType.DMA,
            pltpu.SemaphoreType.DMA,
            pltpu.SemaphoreType.DMA,
        ],
    )
    def k(table_hbm, src_hbm, dst_hbm, out_hbm,
          sidx_v, didx_v, rows_v, zbuf_v, acc_sh, isem, gsem, ssem, zsem):
        c = lax.axis_index("c")
        s = lax.axis_index("s")
        wid = c * NS + s
        zero = jnp.zeros((16,), jnp.float32)

        def fire_idx(g, sel):
            base = wid * ew + g * (KDEPTH * CH)
            out = []
            for j in range(KDEPTH):
                out.append(pltpu.async_copy(
                    src_hbm.at[pl.ds(base + j * CH, CH)],
                    sidx_v.at[sel, j], isem))
                out.append(pltpu.async_copy(
                    dst_hbm.at[pl.ds(base + j * CH, CH)],
                    didx_v.at[sel, j], isem))
            return out

        idesc0 = fire_idx(0, 0)

        def fill_zero(i, _):
            for j in range(f // 16):
                zbuf_v[i, pl.ds(16 * j, 16)] = zero
            return 0

        lax.fori_loop(0, zr, fill_zero, 0)
        zdesc = [
            pltpu.async_copy(
                zbuf_v, acc_sh.at[pl.ds(s * rw + t * zr, zr)], zsem)
            for t in range(5)
        ]
        for dsc in zdesc:
            dsc.wait()
        plsc.subcore_barrier()
        for dsc in idesc0:
            dsc.wait()

        def super_body(g, _):
            sel = g % 2
            oth = 1 - sel

            @pl.when(g > 0)
            def _():
                for j in range(KDEPTH):
                    pltpu.make_async_copy(
                        rows_v.at[oth, j],
                        acc_sh.at[didx_v.at[oth, j]], ssem).wait()

            gdesc = []
            for j in range(KDEPTH):
                gdesc.append(pltpu.async_copy(
                    table_hbm.at[sidx_v.at[sel, j]], rows_v.at[sel, j], gsem))

            @pl.when(g + 1 < nsup)
            def _():
                fire_idx(g + 1, oth)

            for j in range(KDEPTH):
                gdesc[j].wait()
                pltpu.async_copy(
                    rows_v.at[sel, j], acc_sh.at[didx_v.at[sel, j]],
                    ssem, add=True)

            @pl.when(g + 1 < nsup)
            def _():
                base = wid * ew + (g + 1) * (KDEPTH * CH)
                for j in range(KDEPTH):
                    pltpu.make_async_copy(
                        src_hbm.at[pl.ds(base + j * CH, CH)],
                        sidx_v.at[oth, j], isem).wait()
                    pltpu.make_async_copy(
                        dst_hbm.at[pl.ds(base + j * CH, CH)],
                        didx_v.at[oth, j], isem).wait()
            return 0

        lax.fori_loop(0, nsup, super_body, 0)
        lastsel = (nsup - 1) % 2
        for j in range(KDEPTH):
            pltpu.make_async_copy(
                rows_v.at[lastsel, j],
                acc_sh.at[didx_v.at[lastsel, j]], ssem).wait()
        plsc.subcore_barrier()
        pltpu.sync_copy(
            acc_sh.at[pl.ds(s * rw, rw)], out_hbm.at[c, pl.ds(s * rw, rw)]
        )

    return k(table, src1d, dst1d)


def _sc_segsum_m(table, src1d, dst1d, batch1d, dinv1d, nb):
    n, f = table.shape
    e = src1d.shape[0]
    ew = e // NW
    cw = ew // CHM
    rw = NP // NS
    zr = rw // 5
    nsup = cw // KDEPTH
    gr = CHM // 16

    @functools.partial(
        pl.kernel,
        out_type=(jax.ShapeDtypeStruct((NC, NP, f), jnp.float32),
                  jax.ShapeDtypeStruct((NC, NP, nb), jnp.float32)),
        mesh=_mesh(),
        compiler_params=pltpu.CompilerParams(
            use_tc_tiling_on_sc=False, needs_layout_passes=False),
        scratch_types=[
            pltpu.VMEM((2, KDEPTH, CHM), jnp.int32),
            pltpu.VMEM((2, KDEPTH, CHM), jnp.int32),
            pltpu.VMEM((2, KDEPTH, CHM, f), jnp.float32),
            pltpu.VMEM((2, KDEPTH, CHM, nb), jnp.float32),
            pltpu.VMEM((2, KDEPTH, CHM), jnp.int32),
            pltpu.VMEM((zr, f), jnp.float32),
            pltpu.VMEM((n,), jnp.int32),
            pltpu.VMEM((n,), jnp.float32),
            pltpu.MemorySpace.VMEM_SHARED((NP, f), jnp.float32),
            pltpu.MemorySpace.VMEM_SHARED((NP, nb), jnp.float32),
            pltpu.SemaphoreType.DMA,
            pltpu.SemaphoreType.DMA,
            pltpu.SemaphoreType.DMA,
            pltpu.SemaphoreType.DMA,
            pltpu.SemaphoreType.DMA,
            pltpu.SemaphoreType.DMA,
        ],
    )
    def k(table_hbm, src_hbm, dst_hbm, batch_hbm, dinv_hbm,
          out_hbm, mout_hbm,
          sidx_v, didx_v, rows_v, mrows_v, bcol_v, zbuf_v, batch_t, dinv_t,
          acc_sh, macc_sh, isem, gsem, ssem, zsem, msem, tsem):
        c = lax.axis_index("c")
        s = lax.axis_index("s")
        wid = c * NS + s
        zero = jnp.zeros((16,), jnp.float32)
        iota16 = lax.iota(jnp.int32, 16)

        tdesc = [pltpu.async_copy(batch_hbm, batch_t, tsem),
                 pltpu.async_copy(dinv_hbm, dinv_t, tsem)]

        def fire_idx(g, sel):
            base = wid * ew + g * (KDEPTH * CHM)
            out = []
            for j in range(KDEPTH):
                out.append(pltpu.async_copy(
                    src_hbm.at[pl.ds(base + j * CHM, CHM)],
                    sidx_v.at[sel, j], isem))
                out.append(pltpu.async_copy(
                    dst_hbm.at[pl.ds(base + j * CHM, CHM)],
                    didx_v.at[sel, j], isem))
            return out

        idesc0 = fire_idx(0, 0)

        nbv = jnp.full((16,), nb, jnp.int32)

        def zero_mrows(a, j):
            for kk in range(CHM * nb // 16):
                rowv = lax.div(kk * 16 + iota16, nbv)
                colv = lax.rem(kk * 16 + iota16, nbv)
                plsc.store_scatter(mrows_v.at[a, j], [rowv, colv], zero)

        zero_mrows(0, 0)
        zdesc = [
            pltpu.async_copy(
                mrows_v.at[0, 0],
                macc_sh.at[pl.ds(s * rw + t * CHM, CHM)], zsem)
            for t in range(rw // CHM)
        ]
        for a in range(2):
            for j in range(KDEPTH):
                if not (a == 0 and j == 0):
                    zero_mrows(a, j)

        def fill_zero(i, _):
            for j in range(f // 16):
                zbuf_v[i, pl.ds(16 * j, 16)] = zero
            return 0

        lax.fori_loop(0, zr, fill_zero, 0)
        zdesc += [
            pltpu.async_copy(
                zbuf_v, acc_sh.at[pl.ds(s * rw + t * zr, zr)], zsem)
            for t in range(5)
        ]
        for dsc in zdesc:
            dsc.wait()
        plsc.subcore_barrier()
        for dsc in idesc0 + tdesc:
            dsc.wait()

        def super_body(g, _):
            sel = g % 2
            oth = 1 - sel

            @pl.when(g > 0)
            def _():
                for j in range(KDEPTH):
                    pltpu.make_async_copy(
                        rows_v.at[oth, j],
                        acc_sh.at[didx_v.at[oth, j]], ssem).wait()
                    pltpu.make_async_copy(
                        mrows_v.at[oth, j],
                        macc_sh.at[didx_v.at[oth, j]], msem).wait()
                    for i in range(gr):
                        b16 = bcol_v[oth, j, pl.ds(i * 16, 16)]
                        plsc.store_scatter(
                            mrows_v.at[oth, j], [i * 16 + iota16, b16], zero)

            gdesc = []
            for j in range(KDEPTH):
                gdesc.append(pltpu.async_copy(
                    table_hbm.at[sidx_v.at[sel, j]], rows_v.at[sel, j], gsem))

            @pl.when(g + 1 < nsup)
            def _():
                fire_idx(g + 1, oth)

            for j in range(KDEPTH):
                for i in range(gr):
                    srcv = sidx_v[sel, j, pl.ds(i * 16, 16)]
                    b16 = plsc.load_gather(batch_t, [srcv])
                    d16 = plsc.load_gather(dinv_t, [srcv])
                    plsc.store_scatter(
                        mrows_v.at[sel, j], [i * 16 + iota16, b16], d16)
                    bcol_v[sel, j, pl.ds(i * 16, 16)] = b16
                pltpu.async_copy(
                    mrows_v.at[sel, j], macc_sh.at[didx_v.at[sel, j]],
                    msem, add=True)

            for j in range(KDEPTH):
                gdesc[j].wait()
                pltpu.async_copy(
                    rows_v.at[sel, j], acc_sh.at[didx_v.at[sel, j]],
                    ssem, add=True)

            @pl.when(g + 1 < nsup)
            def _():
                base = wid * ew + (g + 1) * (KDEPTH * CHM)
                for j in range(KDEPTH):
                    pltpu.make_async_copy(
                        src_hbm.at[pl.ds(base + j * CHM, CHM)],
                        sidx_v.at[oth, j], isem).wait()
                    pltpu.make_async_copy(
                        dst_hbm.at[pl.ds(base + j * CHM, CHM)],
                        didx_v.at[oth, j], isem).wait()
            return 0

        lax.fori_loop(0, nsup, super_body, 0)
        lastsel = (nsup - 1) % 2
        for j in range(KDEPTH):
            pltpu.make_async_copy(
                rows_v.at[lastsel, j],
                acc_sh.at[didx_v.at[lastsel, j]], ssem).wait()
            pltpu.make_async_copy(
                mrows_v.at[lastsel, j],
                macc_sh.at[didx_v.at[lastsel, j]], msem).wait()
        plsc.subcore_barrier()
        pltpu.sync_copy(
            acc_sh.at[pl.ds(s * rw, rw)], out_hbm.at[c, pl.ds(s * rw, rw)]
        )
        pltpu.sync_copy(
            macc_sh.at[pl.ds(s * rw, rw)], mout_hbm.at[c, pl.ds(s * rw, rw)]
        )

    return k(table, src1d, dst1d, batch1d, dinv1d)


def _tc_call(body, out_shapes, *args):
    return pl.pallas_call(body, out_shape=out_shapes)(*args)


def _k1_body(x_ref, w1_ref, degp_ref, h1p_ref, dinv_ref):
    n = x_ref.shape[0]
    deg = degp_ref[0, :n, 0:1] + degp_ref[1, :n, 0:1] + 1.0
    dinv = lax.rsqrt(deg)
    h = jnp.dot(x_ref[...], w1_ref[...], preferred_element_type=jnp.float32)
    h1p_ref[...] = h * dinv
    dinv_ref[...] = dinv


def _k2_body(a_ref, h1p_ref, dinv_ref, b1_ref, w2_ref, h2p_ref):
    n = h1p_ref.shape[0]
    dinv = dinv_ref[...]
    agg = a_ref[0, :n] + a_ref[1, :n]
    h1 = jnp.maximum(dinv * (agg + h1p_ref[...]) + b1_ref[...], 0.0)
    h2p_ref[...] = jnp.dot(
        h1, w2_ref[...], preferred_element_type=jnp.float32
    ) * dinv


def _k34_body(a_ref, h2p_ref, dinv_ref, b2_ref, batch_ref, w3_ref,
              mp_ref, b3_ref, h4p_ref):
    n = h2p_ref.shape[0]
    dinv = dinv_ref[...]
    agg = a_ref[0, :n] + a_ref[1, :n]
    h2 = dinv * (agg + h2p_ref[...]) + b2_ref[...]
    gids = lax.broadcasted_iota(jnp.int32, (n, 8), 1)
    onehot = (batch_ref[...] == gids).astype(jnp.float32)
    counts = jnp.sum(onehot, axis=0, keepdims=True)
    zsum = lax.dot_general(
        onehot, h2, (((0,), (0,)), ((), ())),
        preferred_element_type=jnp.float32,
    )
    z = zsum / jnp.maximum(counts, 1.0).T
    u = jnp.dot(z, w3_ref[...], preferred_element_type=jnp.float32)
    h3p = jnp.dot(onehot, u, preferred_element_type=jnp.float32) * dinv
    m = mp_ref[0, :n] + mp_ref[1, :n]
    agg3 = jnp.dot(m, u, preferred_element_type=jnp.float32)
    h3 = jnp.maximum(dinv * (agg3 + h3p) + b3_ref[...], 0.0)
    h4p_ref[...] = h3 * dinv


def _k5_body(a_ref, h4p_ref, dinv_ref, b4_ref, w4_ref, out_ref):
    n = h4p_ref.shape[0]
    dinv = dinv_ref[...]
    agg = a_ref[0, :n] + a_ref[1, :n]
    ah = dinv * (agg + h4p_ref[...])
    out_ref[...] = jnp.dot(
        ah, w4_ref[...], preferred_element_type=jnp.float32
    ) + b4_ref[...]


def kernel(x, edge_index, batch, W1, b1, W2, b2, W3, b3, W4, b4):
    n, d = x.shape
    h, l = W1.shape[1], W2.shape[1]

    src1d = edge_index[0]
    dst1d = edge_index[1]
    batch2d = batch.reshape(n, 1)

    degp = _sc_degree(dst1d)

    h1p, dinv = _tc_call(
        _k1_body,
        (jax.ShapeDtypeStruct((n, h), jnp.float32),
         jax.ShapeDtypeStruct((n, 1), jnp.float32)),
        x, W1, degp,
    )
    agg1 = _sc_segsum(h1p, src1d, dst1d)
    h2p = _tc_call(
        _k2_body, jax.ShapeDtypeStruct((n, l), jnp.float32),
        agg1, h1p, dinv, b1, W2,
    )
    agg2, mparts = _sc_segsum_m(
        h2p, src1d, dst1d, batch, dinv.reshape(n), 8)
    h4p = _tc_call(
        _k34_body, jax.ShapeDtypeStruct((n, h), jnp.float32),
        agg2, h2p, dinv, b2, batch2d, W3, mparts, b3,
    )
    agg4 = _sc_segsum(h4p, src1d, dst1d)
    out = _tc_call(
        _k5_body, jax.ShapeDtypeStruct((n, d), jnp.float32),
        agg4, h4p, dinv, b4, W4,
    )
    return out


# speedup vs baseline: 44.9320x; 1.0418x over previous
import functools

import jax
import jax.numpy as jnp
from jax import lax
from jax.experimental import pallas as pl
from jax.experimental.pallas import tpu as pltpu
from jax.experimental.pallas import tpu_sc as plsc

NC, NS = 2, 16
NW = NC * NS
CH = 80
CHM = 80
KDEPTH = 5
NP = 10240

_mesh = functools.partial(
    plsc.VectorSubcoreMesh, core_axis_name="c", subcore_axis_name="s"
)


def _sc_degree(dst1d):
    e = dst1d.shape[0]
    ew = e // NW
    cw = ew // CH
    rw = NP // NS
    zr = rw // 5

    nsup = cw // KDEPTH

    @functools.partial(
        pl.kernel,
        out_type=jax.ShapeDtypeStruct((NC, NP, 16), jnp.float32),
        mesh=_mesh(),
        compiler_params=pltpu.CompilerParams(
            use_tc_tiling_on_sc=False, needs_layout_passes=False),
        scratch_types=[
            pltpu.VMEM((2, KDEPTH, CH), jnp.int32),
            pltpu.VMEM((CH, 16), jnp.float32),
            pltpu.VMEM((zr, 16), jnp.float32),
            pltpu.MemorySpace.VMEM_SHARED((NP, 16), jnp.float32),
            pltpu.SemaphoreType.DMA,
            pltpu.SemaphoreType.DMA,
        ],
    )
    def k(dst_hbm, out_hbm, didx_v, ones_v, zbuf_v, acc_sh, isem, ssem):
        c = lax.axis_index("c")
        s = lax.axis_index("s")
        wid = c * NS + s
        hot = (1 - jnp.minimum(lax.iota(jnp.int32, 16), 1)).astype(jnp.float32)
        zero = jnp.zeros((16,), jnp.float32)

        def fire_idx(g, sel):
            base = wid * ew + g * (KDEPTH * CH)
            return [pltpu.async_copy(
                dst_hbm.at[pl.ds(base + j * CH, CH)],
                didx_v.at[sel, j], isem) for j in range(KDEPTH)]

        idesc0 = fire_idx(0, 0)

        def fill_ones(i, _):
            ones_v[i, :] = hot
            return 0

        lax.fori_loop(0, CH, fill_ones, 0)

        def fill_zero(i, _):
            zbuf_v[i, :] = zero
            return 0

        lax.fori_loop(0, zr, fill_zero, 0)
        for t in range(5):
            pltpu.sync_copy(zbuf_v, acc_sh.at[pl.ds(s * rw + t * zr, zr)])
        plsc.subcore_barrier()
        for dsc in idesc0:
            dsc.wait()

        def super_body(g, _):
            sel = g % 2
            oth = 1 - sel

            @pl.when(g > 0)
            def _():
                for j in range(KDEPTH):
                    pltpu.make_async_copy(
                        ones_v, acc_sh.at[didx_v.at[oth, j]], ssem).wait()

            for j in range(KDEPTH):
                pltpu.async_copy(
                    ones_v, acc_sh.at[didx_v.at[sel, j]], ssem, add=True)

            @pl.when(g + 1 < nsup)
            def _():
                fire_idx(g + 1, oth)
                base = wid * ew + (g + 1) * (KDEPTH * CH)
                for j in range(KDEPTH):
                    pltpu.make_async_copy(
                        dst_hbm.at[pl.ds(base + j * CH, CH)],
                        didx_v.at[oth, j], isem).wait()
            return 0

        lax.fori_loop(0, nsup, super_body, 0)
        lastsel = (nsup - 1) % 2
        for j in range(KDEPTH):
            pltpu.make_async_copy(
                ones_v, acc_sh.at[didx_v.at[lastsel, j]], ssem).wait()
        plsc.subcore_barrier()
        pltpu.sync_copy(
            acc_sh.at[pl.ds(s * rw, rw)], out_hbm.at[c, pl.ds(s * rw, rw)]
        )

    return k(dst1d)


def _sc_segsum(table, src1d, dst1d, fa=None):
    n, f = table.shape
    fa = f if fa is None else fa
    e = src1d.shape[0]
    ew = e // NW
    cw = ew // CH
    rw = NP // NS
    zr = rw // 5

    nsup = cw // KDEPTH

    @functools.partial(
        pl.kernel,
        out_type=jax.ShapeDtypeStruct((NC, NP, fa), jnp.float32),
        mesh=_mesh(),
        compiler_params=pltpu.CompilerParams(
            use_tc_tiling_on_sc=False, needs_layout_passes=False),
        scratch_types=[
            pltpu.VMEM((2, KDEPTH, CH), jnp.int32),
            pltpu.VMEM((2, KDEPTH, CH), jnp.int32),
            pltpu.VMEM((2, KDEPTH, CH, f), jnp.float32),
            pltpu.VMEM((zr, fa), jnp.float32),
            pltpu.MemorySpace.VMEM_SHARED((NP, fa), jnp.float32),
            pltpu.SemaphoreType.DMA,
            pltpu.SemaphoreType.DMA,
            pltpu.SemaphoreType.DMA,
            pltpu.SemaphoreType.DMA,
        ],
    )
    def k(table_hbm, src_hbm, dst_hbm, out_hbm,
          sidx_v, didx_v, rows_v, zbuf_v, acc_sh, isem, gsem, ssem, zsem):
        c = lax.axis_index("c")
        s = lax.axis_index("s")
        wid = c * NS + s
        zero = jnp.zeros((16,), jnp.float32)

        def fire_idx(g, sel):
            base = wid * ew + g * (KDEPTH * CH)
            out = []
            for j in range(KDEPTH):
                out.append(pltpu.async_copy(
                    src_hbm.at[pl.ds(base + j * CH, CH)],
                    sidx_v.at[sel, j], isem))
                out.append(pltpu.async_copy(
                    dst_hbm.at[pl.ds(base + j * CH, CH)],
                    didx_v.at[sel, j], isem))
            return out

        idesc0 = fire_idx(0, 0)

        def srows(a, j):
            if fa == f:
                return rows_v.at[a, j]
            return rows_v.at[a, j, :, pl.ds(0, fa)]

        def fill_zero(i, _):
            for j in range(fa // 16):
                zbuf_v[i, pl.ds(16 * j, 16)] = zero
            return 0

        lax.fori_loop(0, zr, fill_zero, 0)
        zdesc = [
            pltpu.async_copy(
                zbuf_v, acc_sh.at[pl.ds(s * rw + t * zr, zr)], zsem)
            for t in range(5)
        ]
        for dsc in zdesc:
            dsc.wait()
        plsc.subcore_barrier()
        for dsc in idesc0:
            dsc.wait()

        def super_body(g, _):
            sel = g % 2
            oth = 1 - sel

            @pl.when(g > 0)
            def _():
                for j in range(KDEPTH):
                    pltpu.make_async_copy(
                        srows(oth, j),
                        acc_sh.at[didx_v.at[oth, j]], ssem).wait()

            gdesc = []
            for j in range(KDEPTH):
                gdesc.append(pltpu.async_copy(
                    table_hbm.at[sidx_v.at[sel, j]], rows_v.at[sel, j], gsem))

            @pl.when(g + 1 < nsup)
            def _():
                fire_idx(g + 1, oth)

            for j in range(KDEPTH):
                gdesc[j].wait()
                pltpu.async_copy(
                    srows(sel, j), acc_sh.at[didx_v.at[sel, j]],
                    ssem, add=True)

            @pl.when(g + 1 < nsup)
            def _():
                base = wid * ew + (g + 1) * (KDEPTH * CH)
                for j in range(KDEPTH):
                    pltpu.make_async_copy(
                        src_hbm.at[pl.ds(base + j * CH, CH)],
                        sidx_v.at[oth, j], isem).wait()
                    pltpu.make_async_copy(
                        dst_hbm.at[pl.ds(base + j * CH, CH)],
                        didx_v.at[oth, j], isem).wait()
            return 0

        lax.fori_loop(0, nsup, super_body, 0)
        lastsel = (nsup - 1) % 2
        for j in range(KDEPTH):
            pltpu.make_async_copy(
                srows(lastsel, j),
                acc_sh.at[didx_v.at[lastsel, j]], ssem).wait()
        plsc.subcore_barrier()
        pltpu.sync_copy(
            acc_sh.at[pl.ds(s * rw, rw)], out_hbm.at[c, pl.ds(s * rw, rw)]
        )

    return k(table, src1d, dst1d)


def _sc_segsum_m(table, src1d, dst1d, batch1d, dinv1d, nb):
    n, f = table.shape
    e = src1d.shape[0]
    ew = e // NW
    cw = ew // CHM
    rw = NP // NS
    zr = rw // 5
    nsup = cw // KDEPTH
    gr = CHM // 16

    @functools.partial(
        pl.kernel,
        out_type=(jax.ShapeDtypeStruct((NC, NP, f), jnp.float32),
                  jax.ShapeDtypeStruct((NC, NP, nb), jnp.float32)),
        mesh=_mesh(),
        compiler_params=pltpu.CompilerParams(
            use_tc_tiling_on_sc=False, needs_layout_passes=False),
        scratch_types=[
            pltpu.VMEM((2, KDEPTH, CHM), jnp.int32),
            pltpu.VMEM((2, KDEPTH, CHM), jnp.int32),
            pltpu.VMEM((2, KDEPTH, CHM, f), jnp.float32),
            pltpu.VMEM((2, KDEPTH, CHM, nb), jnp.float32),
            pltpu.VMEM((2, KDEPTH, CHM), jnp.int32),
            pltpu.VMEM((zr, f), jnp.float32),
            pltpu.VMEM((n,), jnp.int32),
            pltpu.VMEM((n,), jnp.float32),
            pltpu.MemorySpace.VMEM_SHARED((NP, f), jnp.float32),
            pltpu.MemorySpace.VMEM_SHARED((NP, nb), jnp.float32),
            pltpu.SemaphoreType.DMA,
            pltpu.SemaphoreType.DMA,
            pltpu.SemaphoreType.DMA,
            pltpu.SemaphoreType.DMA,
            pltpu.SemaphoreType.DMA,
            pltpu.SemaphoreType.DMA,
        ],
    )
    def k(table_hbm, src_hbm, dst_hbm, batch_hbm, dinv_hbm,
          out_hbm, mout_hbm,
          sidx_v, didx_v, rows_v, mrows_v, bcol_v, zbuf_v, batch_t, dinv_t,
          acc_sh, macc_sh, isem, gsem, ssem, zsem, msem, tsem):
        c = lax.axis_index("c")
        s = lax.axis_index("s")
        wid = c * NS + s
        zero = jnp.zeros((16,), jnp.float32)
        iota16 = lax.iota(jnp.int32, 16)

        tdesc = [pltpu.async_copy(batch_hbm, batch_t, tsem),
                 pltpu.async_copy(dinv_hbm, dinv_t, tsem)]

        def fire_idx(g, sel):
            base = wid * ew + g * (KDEPTH * CHM)
            out = []
            for j in range(KDEPTH):
                out.append(pltpu.async_copy(
                    src_hbm.at[pl.ds(base + j * CHM, CHM)],
                    sidx_v.at[sel, j], isem))
                out.append(pltpu.async_copy(
                    dst_hbm.at[pl.ds(base + j * CHM, CHM)],
                    didx_v.at[sel, j], isem))
            return out

        idesc0 = fire_idx(0, 0)

        nbv = jnp.full((16,), nb, jnp.int32)

        def zero_mrows(a, j):
            for kk in range(CHM * nb // 16):
                rowv = lax.div(kk * 16 + iota16, nbv)
                colv = lax.rem(kk * 16 + iota16, nbv)
                plsc.store_scatter(mrows_v.at[a, j], [rowv, colv], zero)

        zero_mrows(0, 0)
        zdesc = [
            pltpu.async_copy(
                mrows_v.at[0, 0],
                macc_sh.at[pl.ds(s * rw + t * CHM, CHM)], zsem)
            for t in range(rw // CHM)
        ]
        for a in range(2):
            for j in range(KDEPTH):
                if not (a == 0 and j == 0):
                    zero_mrows(a, j)

        def fill_zero(i, _):
            for j in range(f // 16):
                zbuf_v[i, pl.ds(16 * j, 16)] = zero
            return 0

        lax.fori_loop(0, zr, fill_zero, 0)
        zdesc += [
            pltpu.async_copy(
                zbuf_v, acc_sh.at[pl.ds(s * rw + t * zr, zr)], zsem)
            for t in range(5)
        ]
        for dsc in zdesc:
            dsc.wait()
        plsc.subcore_barrier()
        for dsc in idesc0 + tdesc:
            dsc.wait()

        def super_body(g, _):
            sel = g % 2
            oth = 1 - sel

            @pl.when(g > 0)
            def _():
                for j in range(KDEPTH):
                    pltpu.make_async_copy(
                        rows_v.at[oth, j],
                        acc_sh.at[didx_v.at[oth, j]], ssem).wait()
                    pltpu.make_async_copy(
                        mrows_v.at[oth, j],
                        macc_sh.at[didx_v.at[oth, j]], msem).wait()
                    for i in range(gr):
                        b16 = bcol_v[oth, j, pl.ds(i * 16, 16)]
                        plsc.store_scatter(
                            mrows_v.at[oth, j], [i * 16 + iota16, b16], zero)

            gdesc = []
            for j in range(KDEPTH):
                gdesc.append(pltpu.async_copy(
                    table_hbm.at[sidx_v.at[sel, j]], rows_v.at[sel, j], gsem))

            @pl.when(g + 1 < nsup)
            def _():
                fire_idx(g + 1, oth)

            for j in range(KDEPTH):
                for i in range(gr):
                    srcv = sidx_v[sel, j, pl.ds(i * 16, 16)]
                    b16 = plsc.load_gather(batch_t, [srcv])
                    d16 = plsc.load_gather(dinv_t, [srcv])
                    plsc.store_scatter(
                        mrows_v.at[sel, j], [i * 16 + iota16, b16], d16)
                    bcol_v[sel, j, pl.ds(i * 16, 16)] = b16
                pltpu.async_copy(
                    mrows_v.at[sel, j], macc_sh.at[didx_v.at[sel, j]],
                    msem, add=True)

            for j in range(KDEPTH):
                gdesc[j].wait()
                pltpu.async_copy(
                    rows_v.at[sel, j], acc_sh.at[didx_v.at[sel, j]],
                    ssem, add=True)

            @pl.when(g + 1 < nsup)
            def _():
                base = wid * ew + (g + 1) * (KDEPTH * CHM)
                for j in range(KDEPTH):
                    pltpu.make_async_copy(
                        src_hbm.at[pl.ds(base + j * CHM, CHM)],
                        sidx_v.at[oth, j], isem).wait()
                    pltpu.make_async_copy(
                        dst_hbm.at[pl.ds(base + j * CHM, CHM)],
                        didx_v.at[oth, j], isem).wait()
            return 0

        lax.fori_loop(0, nsup, super_body, 0)
        lastsel = (nsup - 1) % 2
        for j in range(KDEPTH):
            pltpu.make_async_copy(
                rows_v.at[lastsel, j],
                acc_sh.at[didx_v.at[lastsel, j]], ssem).wait()
            pltpu.make_async_copy(
                mrows_v.at[lastsel, j],
                macc_sh.at[didx_v.at[lastsel, j]], msem).wait()
        plsc.subcore_barrier()
        pltpu.sync_copy(
            acc_sh.at[pl.ds(s * rw, rw)], out_hbm.at[c, pl.ds(s * rw, rw)]
        )
        pltpu.sync_copy(
            macc_sh.at[pl.ds(s * rw, rw)], mout_hbm.at[c, pl.ds(s * rw, rw)]
        )

    return k(table, src1d, dst1d, batch1d, dinv1d)


def _tc_call(body, out_shapes, *args):
    return pl.pallas_call(body, out_shape=out_shapes)(*args)


def _slice_body(e_ref, src_ref, dst_ref):
    src_ref[...] = e_ref[0]
    dst_ref[...] = e_ref[1]


def _tc_slice(edge_index):
    e = edge_index.shape[1]
    return pl.pallas_call(
        _slice_body,
        out_shape=(jax.ShapeDtypeStruct((e,), jnp.int32),
                   jax.ShapeDtypeStruct((e,), jnp.int32)),
    )(edge_index)


def _k0_body(x_ref, w1_ref, xw_ref):
    xw_ref[...] = jnp.dot(
        x_ref[...], w1_ref[...], preferred_element_type=jnp.float32)


def _k1_body(xw_ref, degp_ref, h1p_ref, dinv_ref):
    n = xw_ref.shape[0]
    deg = degp_ref[0, :n, 0:1] + degp_ref[1, :n, 0:1] + 1.0
    dinv = lax.rsqrt(deg)
    h1p_ref[...] = xw_ref[...] * dinv
    dinv_ref[...] = dinv


def _k2_body(a_ref, h1p_ref, dinv_ref, b1_ref, w2_ref, h2p_ref):
    n = h1p_ref.shape[0]
    dinv = dinv_ref[...]
    agg = a_ref[0, :n] + a_ref[1, :n]
    h1 = jnp.maximum(dinv * (agg + h1p_ref[...]) + b1_ref[...], 0.0)
    h2p_ref[...] = jnp.dot(
        h1, w2_ref[...], preferred_element_type=jnp.float32
    ) * dinv


def _k34_body(a_ref, h2p_ref, dinv_ref, b2_ref, batch_ref, w3_ref,
              mp_ref, b3_ref, h4p_ref):
    n = h2p_ref.shape[0]
    dinv = dinv_ref[...]
    agg = a_ref[0, :n] + a_ref[1, :n]
    h2 = dinv * (agg + h2p_ref[...]) + b2_ref[...]
    gids = lax.broadcasted_iota(jnp.int32, (n, 8), 1)
    onehot = (batch_ref[...] == gids).astype(jnp.float32)
    counts = jnp.sum(onehot, axis=0, keepdims=True)
    zsum = lax.dot_general(
        onehot, h2, (((0,), (0,)), ((), ())),
        preferred_element_type=jnp.float32,
    )
    z = zsum / jnp.maximum(counts, 1.0).T
    u = jnp.dot(z, w3_ref[...], preferred_element_type=jnp.float32)
    h3p = jnp.dot(onehot, u, preferred_element_type=jnp.float32) * dinv
    m = mp_ref[0, :n] + mp_ref[1, :n]
    agg3 = jnp.dot(m, u, preferred_element_type=jnp.float32)
    h3 = jnp.maximum(dinv * (agg3 + h3p) + b3_ref[...], 0.0)
    h4p_ref[...] = h3 * dinv


def _k5_body(a_ref, h4p_ref, dinv_ref, b4_ref, w4_ref, out_ref):
    n = h4p_ref.shape[0]
    dinv = dinv_ref[...]
    agg = a_ref[0, :n] + a_ref[1, :n]
    ah = dinv * (agg + h4p_ref[...])
    out_ref[...] = jnp.dot(
        ah, w4_ref[...], preferred_element_type=jnp.float32
    ) + b4_ref[...]


def kernel(x, edge_index, batch, W1, b1, W2, b2, W3, b3, W4, b4):
    n, d = x.shape
    h, l = W1.shape[1], W2.shape[1]

    src1d, dst1d = _tc_slice(edge_index)
    batch2d = batch.reshape(n, 1)

    degp = _sc_degree(dst1d)

    xw = _tc_call(
        _k0_body, jax.ShapeDtypeStruct((n, h), jnp.float32), x, W1)
    h1p, dinv = _tc_call(
        _k1_body,
        (jax.ShapeDtypeStruct((n, h), jnp.float32),
         jax.ShapeDtypeStruct((n, 1), jnp.float32)),
        xw, degp,
    )
    agg1 = _sc_segsum(h1p, src1d, dst1d)
    h2p = _tc_call(
        _k2_body, jax.ShapeDtypeStruct((n, l), jnp.float32),
        agg1, h1p, dinv, b1, W2,
    )
    agg2, mparts = _sc_segsum_m(
        h2p, src1d, dst1d, batch, dinv.reshape(n), 8)
    h4p = _tc_call(
        _k34_body, jax.ShapeDtypeStruct((n, h), jnp.float32),
        agg2, h2p, dinv, b2, batch2d, W3, mparts, b3,
    )
    agg4 = _sc_segsum(h4p, src1d, dst1d)
    out = _tc_call(
        _k5_body, jax.ShapeDtypeStruct((n, d), jnp.float32),
        agg4, h4p, dinv, b4, W4,
    )
    return out
